# baseline (speedup 1.0000x reference)
import functools

import jax
import jax.numpy as jnp
import numpy as np
from jax.experimental import pallas as pl
from jax.experimental.pallas import tpu as pltpu
from jax.experimental.pallas import tpu_sc as plsc

_V = 32000
_P_DIM = 256
_HIDDEN = 768
_HEADS = 8
_HEAD_DIM = _HIDDEN // _HEADS
_DEPTH = 8
_K_WIN = 64
_S = 2048
_NV = 256
_TILE = 256
_NT = _S // _TILE

_INTERPRET = False


def _pairwise_dsq(verts):
    G = verts @ verts.T
    n = jnp.diag(G)
    dsq = n[:, None] + n[None, :] - 2.0 * G
    return jnp.clip(dsq, 0.0, None)


def _det3(m):
    return (m[0, 0] * (m[1, 1] * m[2, 2] - m[1, 2] * m[2, 1])
            - m[0, 1] * (m[1, 0] * m[2, 2] - m[1, 2] * m[2, 0])
            + m[0, 2] * (m[1, 0] * m[2, 1] - m[1, 1] * m[2, 0]))


def _det4(m):
    return (m[0, 0] * _det3(m[1:][:, (1, 2, 3)])
            - m[0, 1] * _det3(m[1:][:, (0, 2, 3)])
            + m[0, 2] * _det3(m[1:][:, (0, 1, 3)])
            - m[0, 3] * _det3(m[1:][:, (0, 1, 2)]))


def _cantor_coord(verts):
    dsq = _pairwise_dsq(verts)
    row = dsq[0, 1:]
    gram = 0.5 * (row[:, None] + row[None, :] - dsq[1:, 1:])
    vol_sq = _det4(gram) / 576.0
    volume = jnp.sqrt(jnp.clip(vol_sq, 0.0, None))
    iu = jnp.triu_indices(5, k=1)
    edges = jnp.sqrt(dsq[iu])
    mean_edge = edges.mean()
    std_edge = jnp.std(edges, ddof=1)
    centroid = verts.mean(axis=0)
    dists = jnp.linalg.norm(verts - centroid, axis=-1)
    spread = jnp.std(dists, ddof=1)
    vn = jnp.where(volume > 3.0, jnp.float32(1.0), jax.nn.sigmoid(volume * 10.0))
    er = jax.nn.sigmoid(std_edge / (mean_edge + 1e-06))
    sn = jax.nn.sigmoid(spread)
    seed = jnp.clip(vn * 0.4 + er * 0.3 + sn * 0.3, 1e-06, 1.0 - 1e-06)
    x = seed
    cantor_val = jnp.zeros((), dtype=verts.dtype)
    factor = 0.5
    for _ in range(_DEPTH):
        x_scaled = x * 3.0
        digit = jnp.floor(x_scaled)
        x_frac = x_scaled - digit
        middle_bit = (digit == 2.0).astype(verts.dtype)
        cantor_val = cantor_val + middle_bit * factor
        x = jnp.clip(x_frac + (vn + er + sn) * 0.01, 1e-06, 1.0 - 1e-06)
        factor *= 0.5
    return jnp.clip(cantor_val, 0.0, 1.0)


_NW = 32
_BPW = _S // _NW


def _make_sc_gather():
    mesh = plsc.VectorSubcoreMesh(core_axis_name="c", subcore_axis_name="s")

    @functools.partial(
        pl.kernel, mesh=mesh,
        out_type=jax.ShapeDtypeStruct((_S, 5, _P_DIM), jnp.float32),
        scratch_types=[
            pltpu.VMEM((_BPW,), jnp.int32),
            pltpu.SemaphoreType.DMA,
        ],
    )
    def gather_rows(table_hbm, idx_hbm, out_hbm, idx_v, sem):
        wid = jax.lax.axis_index("s") * 2 + jax.lax.axis_index("c")
        base = wid * _BPW
        pltpu.sync_copy(idx_hbm.at[pl.ds(base, _BPW)], idx_v)
        cps = []
        for g in range(_BPW // 16):
            vec = idx_v[pl.ds(g * 16, 16)]
            for b in range(16):
                cps.append(pltpu.async_copy(
                    table_hbm.at[vec[b]], out_hbm.at[base + g * 16 + b], sem))
        for cp in cps:
            cp.wait()

    return gather_rows


_sc_gather = None


def _get_sc_gather():
    global _sc_gather
    if _sc_gather is None:
        _sc_gather = _make_sc_gather()
    return _sc_gather


def _route_body(crow_ref, rt_ref, tbl_ref):
    crow = crow_ref[0:1, :]
    ar = jax.lax.broadcasted_iota(jnp.int32, (_NV, 1), 0).astype(jnp.float32)
    E = jnp.abs(ar - crow)
    lo = jnp.zeros((_NV, 1), jnp.float32)
    hi = jnp.full((_NV, 1), float(_NV - 1), jnp.float32)
    for _ in range(8):
        mid = jnp.floor((lo + hi) * 0.5)
        cnt = jnp.sum((E <= mid).astype(jnp.float32), axis=1, keepdims=True)
        ge = cnt >= _K_WIN
        hi = jnp.where(ge, mid, hi)
        lo = jnp.where(ge, lo, mid + 1.0)
    D = lo
    nbelow = jnp.sum((E < D).astype(jnp.float32), axis=1, keepdims=True)
    r = _K_WIN - nbelow
    ind = (E == D).astype(jnp.float32)
    acc = ind
    sh = 1
    while sh < _S:
        acc = acc + jnp.concatenate(
            [jnp.zeros((_NV, sh), jnp.float32), acc[:, :_S - sh]], axis=1)
        sh *= 2
    rt_ref[...] = acc - ind
    tbl_ref[...] = jnp.concatenate(
        [ar, D, r, jnp.zeros((_NV, 125), jnp.float32)], axis=1)


def _routing(crow):
    return pl.pallas_call(
        _route_body,
        in_specs=[pl.BlockSpec((8, _S), lambda: (0, 0))],
        out_specs=[
            pl.BlockSpec((_NV, _S), lambda: (0, 0)),
            pl.BlockSpec((_NV, 128), lambda: (0, 0)),
        ],
        out_shape=[
            jax.ShapeDtypeStruct((_NV, _S), jnp.float32),
            jax.ShapeDtypeStruct((_NV, 128), jnp.float32),
        ],
        interpret=_INTERPRET,
    )(crow)


def _proj_body(tv_ref, wf_ref, bf_ref, wq_ref, bq_ref, q_ref, k_ref, v_ref):
    tv = tv_ref[...]
    feats = (tv[:, 0:_P_DIM] + tv[:, _P_DIM:2 * _P_DIM]
             + tv[:, 2 * _P_DIM:3 * _P_DIM] + tv[:, 3 * _P_DIM:4 * _P_DIM]
             + tv[:, 4 * _P_DIM:5 * _P_DIM]) * jnp.float32(0.2)
    x = jnp.dot(feats, wf_ref[...],
                preferred_element_type=jnp.float32) + bf_ref[...]
    qkv = jnp.dot(x, wq_ref[...],
                  preferred_element_type=jnp.float32) + bq_ref[...]
    q_ref[...] = qkv[:, 0:_HIDDEN]
    k_ref[...] = qkv[:, _HIDDEN:2 * _HIDDEN]
    v_ref[...] = qkv[:, 2 * _HIDDEN:3 * _HIDDEN]


def _project(tok_verts_flat, W_feat, b_feat, W_qkv, b_qkv):
    return pl.pallas_call(
        _proj_body,
        grid=(_NT,),
        in_specs=[
            pl.BlockSpec((_TILE, 5 * _P_DIM), lambda i: (i, 0)),
            pl.BlockSpec((_P_DIM, _HIDDEN), lambda i: (0, 0)),
            pl.BlockSpec((1, _HIDDEN), lambda i: (0, 0)),
            pl.BlockSpec((_HIDDEN, 3 * _HIDDEN), lambda i: (0, 0)),
            pl.BlockSpec((1, 3 * _HIDDEN), lambda i: (0, 0)),
        ],
        out_specs=[
            pl.BlockSpec((_TILE, _HIDDEN), lambda i: (i, 0)),
            pl.BlockSpec((_TILE, _HIDDEN), lambda i: (i, 0)),
            pl.BlockSpec((_TILE, _HIDDEN), lambda i: (i, 0)),
        ],
        out_shape=[jax.ShapeDtypeStruct((_S, _HIDDEN), jnp.float32)] * 3,
        interpret=_INTERPRET,
    )(tok_verts_flat, W_feat, b_feat.reshape(1, -1), W_qkv, b_qkv.reshape(1, -1))


def _attn_body(q_ref, k_ref, v_ref, tbl_ref, crow_ref, cfc_ref, rt_ref,
               wo_ref, bo_ref, out_ref, acc_ref):
    cq = cfc_ref[:, 0:1]
    arow = jax.lax.broadcasted_iota(jnp.int32, (1, _NV), 1).astype(jnp.float32)
    onehot = (cq == arow).astype(jnp.float32)
    dr = jnp.dot(onehot, tbl_ref[...],
                 preferred_element_type=jnp.float32,
                 precision=jax.lax.Precision.HIGHEST)
    dq = dr[:, 1:2]
    rq = dr[:, 2:3]
    crow = crow_ref[0:1, :]
    dj = jnp.abs(cq - crow)
    rank = jnp.dot(onehot, rt_ref[...],
                   preferred_element_type=jnp.float32,
                   precision=jax.lax.Precision.HIGHEST)
    mask = (dj < dq) | ((dj == dq) & (rank < rq))
    madd = jnp.where(mask, jnp.float32(0.0), jnp.float32(-1e30))
    scale = jnp.float32(1.0 / np.sqrt(_HEAD_DIM))
    q_all = q_ref[...] * scale
    k_all = k_ref[...]
    v_all = v_ref[...]
    for h in range(_HEADS):
        sl = slice(h * _HEAD_DIM, (h + 1) * _HEAD_DIM)
        qh = q_all[:, sl]
        kh = k_all[:, sl]
        vh = v_all[:, sl]
        s = jax.lax.dot_general(qh, kh, (((1,), (1,)), ((), ())),
                                preferred_element_type=jnp.float32) + madd
        m = jnp.max(s, axis=1, keepdims=True)
        p = jnp.exp(s - m)
        recip = jnp.float32(1.0) / jnp.sum(p, axis=1, keepdims=True)
        o = jnp.dot(p, vh, preferred_element_type=jnp.float32)
        acc_ref[:, sl] = o * recip
    out_ref[...] = jnp.dot(acc_ref[...], wo_ref[...],
                           preferred_element_type=jnp.float32) + bo_ref[...]


def _attention(q, k, v, tbl, crow, cfcol, Rt, W_out, b_out):
    return pl.pallas_call(
        _attn_body,
        grid=(_NT,),
        in_specs=[
            pl.BlockSpec((_TILE, _HIDDEN), lambda i: (i, 0)),
            pl.BlockSpec((_S, _HIDDEN), lambda i: (0, 0)),
            pl.BlockSpec((_S, _HIDDEN), lambda i: (0, 0)),
            pl.BlockSpec((_NV, 128), lambda i: (0, 0)),
            pl.BlockSpec((8, _S), lambda i: (0, 0)),
            pl.BlockSpec((_TILE, 128), lambda i: (i, 0)),
            pl.BlockSpec((_NV, _S), lambda i: (0, 0)),
            pl.BlockSpec((_HIDDEN, _HIDDEN), lambda i: (0, 0)),
            pl.BlockSpec((1, _HIDDEN), lambda i: (0, 0)),
        ],
        out_specs=pl.BlockSpec((_TILE, _HIDDEN), lambda i: (i, 0)),
        out_shape=jax.ShapeDtypeStruct((_S, _HIDDEN), jnp.float32),
        scratch_shapes=[pltpu.VMEM((_TILE, _HIDDEN), jnp.float32)],
        interpret=_INTERPRET,
    )(q, k, v, tbl, crow, cfcol, Rt, W_out, b_out.reshape(1, -1))


def kernel(input_ids, pentachora, W_feat, b_feat, W_qkv, b_qkv, W_out, b_out):
    ids = input_ids.reshape(-1).astype(jnp.int32)
    tok_verts = _get_sc_gather()(pentachora, ids)
    coords = jax.vmap(_cantor_coord)(tok_verts)
    c_int = (coords * jnp.float32(_NV)).astype(jnp.int32)

    cf = c_int.astype(jnp.float32)
    crow = jnp.broadcast_to(cf[None, :], (8, _S))
    cfcol = jnp.broadcast_to(cf[:, None], (_S, 128))
    Rt, tbl = _routing(crow)

    q, k, v = _project(tok_verts.reshape(_S, 5 * _P_DIM),
                       W_feat, b_feat, W_qkv, b_qkv)
    out = _attention(q, k, v, tbl, crow, cfcol, Rt, W_out, b_out)
    return out[None]


# speedup vs baseline: 1.6806x; 1.6806x over previous
import functools

import jax
import jax.numpy as jnp
import numpy as np
from jax.experimental import pallas as pl
from jax.experimental.pallas import tpu as pltpu
from jax.experimental.pallas import tpu_sc as plsc

_V = 32000
_P_DIM = 256
_HIDDEN = 768
_HEADS = 8
_HEAD_DIM = _HIDDEN // _HEADS
_DEPTH = 8
_K_WIN = 64
_S = 2048
_NV = 256
_TILE = 256
_NT = _S // _TILE

_INTERPRET = False


def _pairwise_dsq(verts):
    G = verts @ verts.T
    n = jnp.diag(G)
    dsq = n[:, None] + n[None, :] - 2.0 * G
    return jnp.clip(dsq, 0.0, None)


def _det3(m):
    return (m[0, 0] * (m[1, 1] * m[2, 2] - m[1, 2] * m[2, 1])
            - m[0, 1] * (m[1, 0] * m[2, 2] - m[1, 2] * m[2, 0])
            + m[0, 2] * (m[1, 0] * m[2, 1] - m[1, 1] * m[2, 0]))


def _det4(m):
    return (m[0, 0] * _det3(m[1:][:, (1, 2, 3)])
            - m[0, 1] * _det3(m[1:][:, (0, 2, 3)])
            + m[0, 2] * _det3(m[1:][:, (0, 1, 3)])
            - m[0, 3] * _det3(m[1:][:, (0, 1, 2)]))


def _cantor_coord(verts):
    dsq = _pairwise_dsq(verts)
    row = dsq[0, 1:]
    gram = 0.5 * (row[:, None] + row[None, :] - dsq[1:, 1:])
    vol_sq = _det4(gram) / 576.0
    volume = jnp.sqrt(jnp.clip(vol_sq, 0.0, None))
    iu = jnp.triu_indices(5, k=1)
    edges = jnp.sqrt(dsq[iu])
    mean_edge = edges.mean()
    std_edge = jnp.std(edges, ddof=1)
    centroid = verts.mean(axis=0)
    dists = jnp.linalg.norm(verts - centroid, axis=-1)
    spread = jnp.std(dists, ddof=1)
    vn = jnp.where(volume > 3.0, jnp.float32(1.0), jax.nn.sigmoid(volume * 10.0))
    er = jax.nn.sigmoid(std_edge / (mean_edge + 1e-06))
    sn = jax.nn.sigmoid(spread)
    seed = jnp.clip(vn * 0.4 + er * 0.3 + sn * 0.3, 1e-06, 1.0 - 1e-06)
    x = seed
    cantor_val = jnp.zeros((), dtype=verts.dtype)
    factor = 0.5
    for _ in range(_DEPTH):
        x_scaled = x * 3.0
        digit = jnp.floor(x_scaled)
        x_frac = x_scaled - digit
        middle_bit = (digit == 2.0).astype(verts.dtype)
        cantor_val = cantor_val + middle_bit * factor
        x = jnp.clip(x_frac + (vn + er + sn) * 0.01, 1e-06, 1.0 - 1e-06)
        factor *= 0.5
    return jnp.clip(cantor_val, 0.0, 1.0)


_NW = 32
_BPW = _S // _NW


def _make_sc_gather():
    mesh = plsc.VectorSubcoreMesh(core_axis_name="c", subcore_axis_name="s")

    @functools.partial(
        pl.kernel, mesh=mesh,
        out_type=jax.ShapeDtypeStruct((_S, 5, _P_DIM), jnp.float32),
        scratch_types=[
            pltpu.VMEM((_BPW,), jnp.int32),
            pltpu.VMEM((_BPW, 5, _P_DIM), jnp.float32),
            pltpu.SemaphoreType.DMA,
        ],
        compiler_params=pltpu.CompilerParams(use_tc_tiling_on_sc=False),
    )
    def gather_rows(table_hbm, idx_hbm, out_hbm, idx_v, rows_v, sem):
        wid = jax.lax.axis_index("s") * 2 + jax.lax.axis_index("c")
        base = wid * _BPW
        pltpu.sync_copy(idx_hbm.at[pl.ds(base, _BPW)], idx_v)
        pltpu.async_copy(table_hbm.at[idx_v], rows_v, sem).wait()
        pltpu.sync_copy(rows_v, out_hbm.at[pl.ds(base, _BPW)])

    return gather_rows


_sc_gather = None


def _get_sc_gather():
    global _sc_gather
    if _sc_gather is None:
        _sc_gather = _make_sc_gather()
    return _sc_gather


def _route_body(crow_ref, rt_ref, tbl_ref):
    crow = crow_ref[0:1, :]
    ar = jax.lax.broadcasted_iota(jnp.int32, (_NV, 1), 0).astype(jnp.float32)
    E = jnp.abs(ar - crow)
    lo = jnp.zeros((_NV, 1), jnp.float32)
    hi = jnp.full((_NV, 1), float(_NV - 1), jnp.float32)
    for _ in range(8):
        mid = jnp.floor((lo + hi) * 0.5)
        cnt = jnp.sum((E <= mid).astype(jnp.float32), axis=1, keepdims=True)
        ge = cnt >= _K_WIN
        hi = jnp.where(ge, mid, hi)
        lo = jnp.where(ge, lo, mid + 1.0)
    D = lo
    nbelow = jnp.sum((E < D).astype(jnp.float32), axis=1, keepdims=True)
    r = _K_WIN - nbelow
    ind = (E == D).astype(jnp.float32)
    acc = ind
    sh = 1
    while sh < _S:
        acc = acc + jnp.concatenate(
            [jnp.zeros((_NV, sh), jnp.float32), acc[:, :_S - sh]], axis=1)
        sh *= 2
    rt_ref[...] = acc - ind
    tbl_ref[...] = jnp.concatenate(
        [ar, D, r, jnp.zeros((_NV, 125), jnp.float32)], axis=1)


def _routing(crow):
    return pl.pallas_call(
        _route_body,
        in_specs=[pl.BlockSpec((8, _S), lambda: (0, 0))],
        out_specs=[
            pl.BlockSpec((_NV, _S), lambda: (0, 0)),
            pl.BlockSpec((_NV, 128), lambda: (0, 0)),
        ],
        out_shape=[
            jax.ShapeDtypeStruct((_NV, _S), jnp.float32),
            jax.ShapeDtypeStruct((_NV, 128), jnp.float32),
        ],
        interpret=_INTERPRET,
    )(crow)


def _proj_body(tv_ref, wf_ref, bf_ref, wq_ref, bq_ref, q_ref, k_ref, v_ref):
    tv = tv_ref[...]
    feats = (tv[:, 0:_P_DIM] + tv[:, _P_DIM:2 * _P_DIM]
             + tv[:, 2 * _P_DIM:3 * _P_DIM] + tv[:, 3 * _P_DIM:4 * _P_DIM]
             + tv[:, 4 * _P_DIM:5 * _P_DIM]) * jnp.float32(0.2)
    x = jnp.dot(feats, wf_ref[...],
                preferred_element_type=jnp.float32) + bf_ref[...]
    qkv = jnp.dot(x, wq_ref[...],
                  preferred_element_type=jnp.float32) + bq_ref[...]
    q_ref[...] = qkv[:, 0:_HIDDEN]
    k_ref[...] = qkv[:, _HIDDEN:2 * _HIDDEN]
    v_ref[...] = qkv[:, 2 * _HIDDEN:3 * _HIDDEN]


def _project(tok_verts_flat, W_feat, b_feat, W_qkv, b_qkv):
    return pl.pallas_call(
        _proj_body,
        grid=(_NT,),
        in_specs=[
            pl.BlockSpec((_TILE, 5 * _P_DIM), lambda i: (i, 0)),
            pl.BlockSpec((_P_DIM, _HIDDEN), lambda i: (0, 0)),
            pl.BlockSpec((1, _HIDDEN), lambda i: (0, 0)),
            pl.BlockSpec((_HIDDEN, 3 * _HIDDEN), lambda i: (0, 0)),
            pl.BlockSpec((1, 3 * _HIDDEN), lambda i: (0, 0)),
        ],
        out_specs=[
            pl.BlockSpec((_TILE, _HIDDEN), lambda i: (i, 0)),
            pl.BlockSpec((_TILE, _HIDDEN), lambda i: (i, 0)),
            pl.BlockSpec((_TILE, _HIDDEN), lambda i: (i, 0)),
        ],
        out_shape=[jax.ShapeDtypeStruct((_S, _HIDDEN), jnp.float32)] * 3,
        interpret=_INTERPRET,
    )(tok_verts_flat, W_feat, b_feat.reshape(1, -1), W_qkv, b_qkv.reshape(1, -1))


def _attn_body(q_ref, k_ref, v_ref, tbl_ref, crow_ref, cfc_ref, rt_ref,
               wo_ref, bo_ref, out_ref, acc_ref):
    cq = cfc_ref[:, 0:1]
    arow = jax.lax.broadcasted_iota(jnp.int32, (1, _NV), 1).astype(jnp.float32)
    onehot = (cq == arow).astype(jnp.float32)
    dr = jnp.dot(onehot, tbl_ref[...],
                 preferred_element_type=jnp.float32,
                 precision=jax.lax.Precision.HIGHEST)
    dq = dr[:, 1:2]
    rq = dr[:, 2:3]
    crow = crow_ref[0:1, :]
    dj = jnp.abs(cq - crow)
    rank = jnp.dot(onehot, rt_ref[...],
                   preferred_element_type=jnp.float32,
                   precision=jax.lax.Precision.HIGHEST)
    mask = (dj < dq) | ((dj == dq) & (rank < rq))
    madd = jnp.where(mask, jnp.float32(0.0), jnp.float32(-1e30))
    scale = jnp.float32(1.0 / np.sqrt(_HEAD_DIM))
    q_all = q_ref[...] * scale
    k_all = k_ref[...]
    v_all = v_ref[...]
    for h in range(_HEADS):
        sl = slice(h * _HEAD_DIM, (h + 1) * _HEAD_DIM)
        qh = q_all[:, sl]
        kh = k_all[:, sl]
        vh = v_all[:, sl]
        s = jax.lax.dot_general(qh, kh, (((1,), (1,)), ((), ())),
                                preferred_element_type=jnp.float32) + madd
        m = jnp.max(s, axis=1, keepdims=True)
        p = jnp.exp(s - m)
        recip = jnp.float32(1.0) / jnp.sum(p, axis=1, keepdims=True)
        o = jnp.dot(p, vh, preferred_element_type=jnp.float32)
        acc_ref[:, sl] = o * recip
    out_ref[...] = jnp.dot(acc_ref[...], wo_ref[...],
                           preferred_element_type=jnp.float32) + bo_ref[...]


def _attention(q, k, v, tbl, crow, cfcol, Rt, W_out, b_out):
    return pl.pallas_call(
        _attn_body,
        grid=(_NT,),
        in_specs=[
            pl.BlockSpec((_TILE, _HIDDEN), lambda i: (i, 0)),
            pl.BlockSpec((_S, _HIDDEN), lambda i: (0, 0)),
            pl.BlockSpec((_S, _HIDDEN), lambda i: (0, 0)),
            pl.BlockSpec((_NV, 128), lambda i: (0, 0)),
            pl.BlockSpec((8, _S), lambda i: (0, 0)),
            pl.BlockSpec((_TILE, 128), lambda i: (i, 0)),
            pl.BlockSpec((_NV, _S), lambda i: (0, 0)),
            pl.BlockSpec((_HIDDEN, _HIDDEN), lambda i: (0, 0)),
            pl.BlockSpec((1, _HIDDEN), lambda i: (0, 0)),
        ],
        out_specs=pl.BlockSpec((_TILE, _HIDDEN), lambda i: (i, 0)),
        out_shape=jax.ShapeDtypeStruct((_S, _HIDDEN), jnp.float32),
        scratch_shapes=[pltpu.VMEM((_TILE, _HIDDEN), jnp.float32)],
        interpret=_INTERPRET,
    )(q, k, v, tbl, crow, cfcol, Rt, W_out, b_out.reshape(1, -1))


def kernel(input_ids, pentachora, W_feat, b_feat, W_qkv, b_qkv, W_out, b_out):
    ids = input_ids.reshape(-1).astype(jnp.int32)
    tok_verts = _get_sc_gather()(pentachora, ids)
    coords = jax.vmap(_cantor_coord)(tok_verts)
    c_int = (coords * jnp.float32(_NV)).astype(jnp.int32)

    cf = c_int.astype(jnp.float32)
    crow = jnp.broadcast_to(cf[None, :], (8, _S))
    cfcol = jnp.broadcast_to(cf[:, None], (_S, 128))
    Rt, tbl = _routing(crow)

    q, k, v = _project(tok_verts.reshape(_S, 5 * _P_DIM),
                       W_feat, b_feat, W_qkv, b_qkv)
    out = _attention(q, k, v, tbl, crow, cfcol, Rt, W_out, b_out)
    return out[None]


# speedup vs baseline: 3.5290x; 2.0998x over previous
import functools

import jax
import jax.numpy as jnp
import numpy as np
from jax.experimental import pallas as pl
from jax.experimental.pallas import tpu as pltpu
from jax.experimental.pallas import tpu_sc as plsc

_V = 32000
_P_DIM = 256
_HIDDEN = 768
_HEADS = 8
_HEAD_DIM = _HIDDEN // _HEADS
_DEPTH = 8
_K_WIN = 64
_S = 2048
_NV = 256
_TILE = 256
_NT = _S // _TILE

_INTERPRET = False


def _pairwise_dsq(verts):
    G = verts @ verts.T
    n = jnp.diag(G)
    dsq = n[:, None] + n[None, :] - 2.0 * G
    return jnp.clip(dsq, 0.0, None)


def _det3(m):
    return (m[0, 0] * (m[1, 1] * m[2, 2] - m[1, 2] * m[2, 1])
            - m[0, 1] * (m[1, 0] * m[2, 2] - m[1, 2] * m[2, 0])
            + m[0, 2] * (m[1, 0] * m[2, 1] - m[1, 1] * m[2, 0]))


def _det4(m):
    return (m[0, 0] * _det3(m[1:][:, (1, 2, 3)])
            - m[0, 1] * _det3(m[1:][:, (0, 2, 3)])
            + m[0, 2] * _det3(m[1:][:, (0, 1, 3)])
            - m[0, 3] * _det3(m[1:][:, (0, 1, 2)]))


def _cantor_coord(verts):
    dsq = _pairwise_dsq(verts)
    row = dsq[0, 1:]
    gram = 0.5 * (row[:, None] + row[None, :] - dsq[1:, 1:])
    vol_sq = _det4(gram) / 576.0
    volume = jnp.sqrt(jnp.clip(vol_sq, 0.0, None))
    iu = jnp.triu_indices(5, k=1)
    edges = jnp.sqrt(dsq[iu])
    mean_edge = edges.mean()
    std_edge = jnp.std(edges, ddof=1)
    centroid = verts.mean(axis=0)
    dists = jnp.linalg.norm(verts - centroid, axis=-1)
    spread = jnp.std(dists, ddof=1)
    vn = jnp.where(volume > 3.0, jnp.float32(1.0), jax.nn.sigmoid(volume * 10.0))
    er = jax.nn.sigmoid(std_edge / (mean_edge + 1e-06))
    sn = jax.nn.sigmoid(spread)
    seed = jnp.clip(vn * 0.4 + er * 0.3 + sn * 0.3, 1e-06, 1.0 - 1e-06)
    x = seed
    cantor_val = jnp.zeros((), dtype=verts.dtype)
    factor = 0.5
    for _ in range(_DEPTH):
        x_scaled = x * 3.0
        digit = jnp.floor(x_scaled)
        x_frac = x_scaled - digit
        middle_bit = (digit == 2.0).astype(verts.dtype)
        cantor_val = cantor_val + middle_bit * factor
        x = jnp.clip(x_frac + (vn + er + sn) * 0.01, 1e-06, 1.0 - 1e-06)
        factor *= 0.5
    return jnp.clip(cantor_val, 0.0, 1.0)


_NW = 32
_BPW = _S // _NW


def _make_sc_gather():
    mesh = plsc.VectorSubcoreMesh(core_axis_name="c", subcore_axis_name="s")

    @functools.partial(
        pl.kernel, mesh=mesh,
        out_type=jax.ShapeDtypeStruct((_S, 5, _P_DIM), jnp.float32),
        scratch_types=[
            pltpu.VMEM((_BPW,), jnp.int32),
            pltpu.VMEM((_BPW, 5, _P_DIM), jnp.float32),
            pltpu.SemaphoreType.DMA,
        ],
        compiler_params=pltpu.CompilerParams(use_tc_tiling_on_sc=False),
    )
    def gather_rows(table_hbm, idx_hbm, out_hbm, idx_v, rows_v, sem):
        wid = jax.lax.axis_index("s") * 2 + jax.lax.axis_index("c")
        base = wid * _BPW
        pltpu.sync_copy(idx_hbm.at[pl.ds(base, _BPW)], idx_v)
        pltpu.async_copy(table_hbm.at[idx_v], rows_v, sem).wait()
        pltpu.sync_copy(rows_v, out_hbm.at[pl.ds(base, _BPW)])

    return gather_rows


_sc_gather = None


def _get_sc_gather():
    global _sc_gather
    if _sc_gather is None:
        _sc_gather = _make_sc_gather()
    return _sc_gather


def _route_body(crow_ref, rt_ref, tbl_ref):
    crow = crow_ref[0:1, :]
    ar = jax.lax.broadcasted_iota(jnp.int32, (_NV, 1), 0).astype(jnp.float32)
    E = jnp.abs(ar - crow)
    lo = jnp.zeros((_NV, 1), jnp.float32)
    hi = jnp.full((_NV, 1), float(_NV - 1), jnp.float32)
    for _ in range(8):
        mid = jnp.floor((lo + hi) * 0.5)
        cnt = jnp.sum((E <= mid).astype(jnp.float32), axis=1, keepdims=True)
        ge = cnt >= _K_WIN
        hi = jnp.where(ge, mid, hi)
        lo = jnp.where(ge, lo, mid + 1.0)
    D = lo
    nbelow = jnp.sum((E < D).astype(jnp.float32), axis=1, keepdims=True)
    r = _K_WIN - nbelow
    ind = (E == D).astype(jnp.float32)
    acc = ind
    sh = 1
    while sh < _S:
        acc = acc + jnp.concatenate(
            [jnp.zeros((_NV, sh), jnp.float32), acc[:, :_S - sh]], axis=1)
        sh *= 2
    rt_ref[...] = acc - ind
    tbl_ref[...] = jnp.concatenate(
        [ar, D, r, jnp.zeros((_NV, 125), jnp.float32)], axis=1)


def _routing(crow):
    return pl.pallas_call(
        _route_body,
        in_specs=[pl.BlockSpec((8, _S), lambda: (0, 0))],
        out_specs=[
            pl.BlockSpec((_NV, _S), lambda: (0, 0)),
            pl.BlockSpec((_NV, 128), lambda: (0, 0)),
        ],
        out_shape=[
            jax.ShapeDtypeStruct((_NV, _S), jnp.float32),
            jax.ShapeDtypeStruct((_NV, 128), jnp.float32),
        ],
        interpret=_INTERPRET,
    )(crow)


def _proj_body(tv_ref, wf_ref, bf_ref, wq_ref, bq_ref, q_ref, k_ref, v_ref):
    tv = tv_ref[...]
    feats = (tv[:, 0:_P_DIM] + tv[:, _P_DIM:2 * _P_DIM]
             + tv[:, 2 * _P_DIM:3 * _P_DIM] + tv[:, 3 * _P_DIM:4 * _P_DIM]
             + tv[:, 4 * _P_DIM:5 * _P_DIM]) * jnp.float32(0.2)
    x = jnp.dot(feats, wf_ref[...],
                preferred_element_type=jnp.float32) + bf_ref[...]
    qkv = jnp.dot(x, wq_ref[...],
                  preferred_element_type=jnp.float32) + bq_ref[...]
    q_ref[...] = qkv[:, 0:_HIDDEN]
    k_ref[...] = qkv[:, _HIDDEN:2 * _HIDDEN]
    v_ref[...] = qkv[:, 2 * _HIDDEN:3 * _HIDDEN]


def _project(tok_verts_flat, W_feat, b_feat, W_qkv, b_qkv):
    return pl.pallas_call(
        _proj_body,
        grid=(_NT,),
        in_specs=[
            pl.BlockSpec((_TILE, 5 * _P_DIM), lambda i: (i, 0)),
            pl.BlockSpec((_P_DIM, _HIDDEN), lambda i: (0, 0)),
            pl.BlockSpec((1, _HIDDEN), lambda i: (0, 0)),
            pl.BlockSpec((_HIDDEN, 3 * _HIDDEN), lambda i: (0, 0)),
            pl.BlockSpec((1, 3 * _HIDDEN), lambda i: (0, 0)),
        ],
        out_specs=[
            pl.BlockSpec((_TILE, _HIDDEN), lambda i: (i, 0)),
            pl.BlockSpec((_TILE, _HIDDEN), lambda i: (i, 0)),
            pl.BlockSpec((_TILE, _HIDDEN), lambda i: (i, 0)),
        ],
        out_shape=[jax.ShapeDtypeStruct((_S, _HIDDEN), jnp.float32)] * 3,
        interpret=_INTERPRET,
    )(tok_verts_flat, W_feat, b_feat.reshape(1, -1), W_qkv, b_qkv.reshape(1, -1))


def _attn_body(q_ref, k_ref, v_ref, tbl_ref, crow_ref, cfc_ref, rt_ref,
               wo_ref, bo_ref, out_ref, acc_ref):
    cq = cfc_ref[:, 0:1]
    arow = jax.lax.broadcasted_iota(jnp.int32, (1, _NV), 1).astype(jnp.float32)
    onehot = (cq == arow).astype(jnp.float32)
    dr = jnp.dot(onehot, tbl_ref[...],
                 preferred_element_type=jnp.float32,
                 precision=jax.lax.Precision.HIGHEST)
    dq = dr[:, 1:2]
    rq = dr[:, 2:3]
    crow = crow_ref[0:1, :]
    dj = jnp.abs(cq - crow)
    rank = jnp.dot(onehot, rt_ref[...],
                   preferred_element_type=jnp.float32,
                   precision=jax.lax.Precision.HIGHEST)
    mask = (dj < dq) | ((dj == dq) & (rank < rq))
    madd = jnp.where(mask, jnp.float32(0.0), jnp.float32(-1e30))
    scale = jnp.float32(1.0 / np.sqrt(_HEAD_DIM))
    q_all = q_ref[...] * scale
    k_all = k_ref[...]
    v_all = v_ref[...]
    for h in range(_HEADS):
        sl = slice(h * _HEAD_DIM, (h + 1) * _HEAD_DIM)
        qh = q_all[:, sl]
        kh = k_all[:, sl]
        vh = v_all[:, sl]
        s = jax.lax.dot_general(qh, kh, (((1,), (1,)), ((), ())),
                                preferred_element_type=jnp.float32) + madd
        m = jnp.max(s, axis=1, keepdims=True)
        p = jnp.exp(s - m)
        recip = jnp.float32(1.0) / jnp.sum(p, axis=1, keepdims=True)
        o = jnp.dot(p, vh, preferred_element_type=jnp.float32)
        acc_ref[:, sl] = o * recip
    out_ref[...] = jnp.dot(acc_ref[...], wo_ref[...],
                           preferred_element_type=jnp.float32) + bo_ref[...]


def _attention(q, k, v, tbl, crow, cfcol, Rt, W_out, b_out):
    return pl.pallas_call(
        _attn_body,
        grid=(_NT,),
        in_specs=[
            pl.BlockSpec((_TILE, _HIDDEN), lambda i: (i, 0)),
            pl.BlockSpec((_S, _HIDDEN), lambda i: (0, 0)),
            pl.BlockSpec((_S, _HIDDEN), lambda i: (0, 0)),
            pl.BlockSpec((_NV, 128), lambda i: (0, 0)),
            pl.BlockSpec((8, _S), lambda i: (0, 0)),
            pl.BlockSpec((_TILE, 128), lambda i: (i, 0)),
            pl.BlockSpec((_NV, _S), lambda i: (0, 0)),
            pl.BlockSpec((_HIDDEN, _HIDDEN), lambda i: (0, 0)),
            pl.BlockSpec((1, _HIDDEN), lambda i: (0, 0)),
        ],
        out_specs=pl.BlockSpec((_TILE, _HIDDEN), lambda i: (i, 0)),
        out_shape=jax.ShapeDtypeStruct((_S, _HIDDEN), jnp.float32),
        scratch_shapes=[pltpu.VMEM((_TILE, _HIDDEN), jnp.float32)],
        interpret=_INTERPRET,
    )(q, k, v, tbl, crow, cfcol, Rt, W_out, b_out.reshape(1, -1))


def kernel(input_ids, pentachora, W_feat, b_feat, W_qkv, b_qkv, W_out, b_out):
    ids = input_ids.reshape(-1).astype(jnp.int32)
    tok_verts = pentachora[ids]
    coords = jax.vmap(_cantor_coord)(tok_verts)
    c_int = (coords * jnp.float32(_NV)).astype(jnp.int32)

    cf = c_int.astype(jnp.float32)
    crow = jnp.broadcast_to(cf[None, :], (8, _S))
    cfcol = jnp.broadcast_to(cf[:, None], (_S, 128))
    Rt, tbl = _routing(crow)

    q, k, v = _project(tok_verts.reshape(_S, 5 * _P_DIM),
                       W_feat, b_feat, W_qkv, b_qkv)
    out = _attention(q, k, v, tbl, crow, cfcol, Rt, W_out, b_out)
    return out[None]
